# baseline (speedup 1.0000x reference)
import functools

import jax
import jax.numpy as jnp
from jax import lax
from jax.experimental import pallas as pl
from jax.experimental.pallas import tpu as pltpu
from jax.experimental.pallas import tpu_sc as plsc

N = 10000
H = 128
NPAD = 10240
E = 320000
K = 128
NTILES = 32
CHUNKS = 80
NH = 2
HC = CHUNKS // NH
EP_TILE = K * CHUNKS
EPAD = EP_TILE * NTILES
RPT = NPAD // 16
BLK = 512
GRID = NPAD // BLK

_MESH = dict(core_axis_name="c", subcore_axis_name="s")


def _sc_deg_body(col_hbm, ones_hbm, zeros_hbm, out_hbm, coli_v, ones_v, acc_sh):
    c = lax.axis_index("c")
    s = lax.axis_index("s")
    wid = s * 2 + c
    rbase = pl.multiple_of(s * RPT, RPT)

    pltpu.sync_copy(zeros_hbm.at[pl.ds(rbase, RPT)], acc_sh.at[pl.ds(rbase, RPT)])
    pltpu.sync_copy(ones_hbm, ones_v)
    plsc.subcore_barrier()

    ebase = pl.multiple_of(wid * EP_TILE, K)

    def body(j, carry):
        off = pl.multiple_of(ebase + j * K, K)
        pltpu.sync_copy(col_hbm.at[pl.ds(off, K)], coli_v)
        pltpu.sync_copy(ones_v, acc_sh.at[coli_v], add=True)
        return carry

    lax.fori_loop(0, CHUNKS, body, 0)
    plsc.subcore_barrier()
    pltpu.sync_copy(acc_sh.at[pl.ds(rbase, RPT)], out_hbm.at[c, pl.ds(rbase, RPT)])


_sc_deg = functools.partial(
    pl.kernel,
    mesh=plsc.VectorSubcoreMesh(**_MESH),
    out_type=jax.ShapeDtypeStruct((2, NPAD, H), jnp.float32),
    scratch_types=[
        pltpu.VMEM((K,), jnp.int32),
        pltpu.VMEM((K, H), jnp.float32),
        pltpu.VMEM_SHARED((NPAD, H), jnp.float32),
    ],
)(_sc_deg_body)


def _sc_scatter_body(u_hbm, row_hbm, col_hbm, zeros_hbm, out_hbm,
                     rowi0, rowi1, coli0, coli1, buf0, buf1, acc_sh,
                     gsem0):
    c = lax.axis_index("c")
    s = lax.axis_index("s")
    wid = s * 2 + c
    rbase = pl.multiple_of(s * RPT, RPT)

    pltpu.sync_copy(zeros_hbm.at[pl.ds(rbase, RPT)], acc_sh.at[pl.ds(rbase, RPT)])
    plsc.subcore_barrier()

    ebase = pl.multiple_of(wid * EP_TILE, K)

    def fire(jj, rowi, buf):
        pltpu.sync_copy(row_hbm.at[pl.ds(pl.multiple_of(ebase + jj * K, K), K)], rowi)
        pltpu.async_copy(u_hbm.at[rowi], buf, gsem0)

    def drain(rowi, buf):
        pltpu.make_async_copy(u_hbm.at[rowi], buf, gsem0).wait()

    fire(0, rowi0, buf0)

    def step(jj, rowi, coli, buf, nrowi, nbuf):
        drain(rowi, buf)

        @pl.when(jj + 1 < CHUNKS)
        def _():
            fire(jj + 1, nrowi, nbuf)

        pltpu.sync_copy(col_hbm.at[pl.ds(pl.multiple_of(ebase + jj * K, K), K)], coli)
        pltpu.sync_copy(buf, acc_sh.at[coli], add=True)

    def body(i, carry):
        j0 = pl.multiple_of(i * 2, 2)
        step(j0, rowi0, coli0, buf0, rowi1, buf1)
        step(j0 + 1, rowi1, coli1, buf1, rowi0, buf0)
        return carry

    lax.fori_loop(0, CHUNKS // 2, body, 0)
    plsc.subcore_barrier()
    pltpu.sync_copy(acc_sh.at[pl.ds(rbase, RPT)], out_hbm.at[c, pl.ds(rbase, RPT)])


_sc_scatter = functools.partial(
    pl.kernel,
    mesh=plsc.VectorSubcoreMesh(**_MESH),
    out_type=jax.ShapeDtypeStruct((2, NPAD, H), jnp.float32),
    scratch_types=[
        pltpu.VMEM((K,), jnp.int32),
        pltpu.VMEM((K,), jnp.int32),
        pltpu.VMEM((K,), jnp.int32),
        pltpu.VMEM((K,), jnp.int32),
        pltpu.VMEM((K, H), jnp.float32),
        pltpu.VMEM((K, H), jnp.float32),
        pltpu.VMEM_SHARED((NPAD, H), jnp.float32),
        pltpu.SemaphoreType.DMA,
    ],
)(_sc_scatter_body)


def _dis(degp_blk):
    d = jnp.sum(jnp.sum(degp_blk, axis=0), axis=1, keepdims=True) * (1.0 / H)
    return lax.rsqrt(1.0 + d)


def _tc_enc_body(x_ref, degp_ref, W1_ref, b1_ref, W2_ref, b2_ref, Wc_ref, u_ref):
    dis = _dis(degp_ref[...])
    h = jnp.dot(x_ref[...], W1_ref[...], preferred_element_type=jnp.float32)
    h = jax.nn.relu(h + b1_ref[...])
    h = jnp.dot(h, W2_ref[...], preferred_element_type=jnp.float32) + b2_ref[...]
    u_ref[...] = dis * jnp.dot(h, Wc_ref[...], preferred_element_type=jnp.float32)


def _tc_enc(xp, degp, W1, b1, W2, b2, Wc0):
    return pl.pallas_call(
        _tc_enc_body,
        grid=(GRID,),
        in_specs=[
            pl.BlockSpec((BLK, H), lambda i: (i, 0)),
            pl.BlockSpec((2, BLK, H), lambda i: (0, i, 0)),
            pl.BlockSpec((H, H), lambda i: (0, 0)),
            pl.BlockSpec((1, H), lambda i: (0, 0)),
            pl.BlockSpec((H, H), lambda i: (0, 0)),
            pl.BlockSpec((1, H), lambda i: (0, 0)),
            pl.BlockSpec((H, H), lambda i: (0, 0)),
        ],
        out_specs=pl.BlockSpec((BLK, H), lambda i: (i, 0)),
        out_shape=jax.ShapeDtypeStruct((NPAD, H), jnp.float32),
    )(xp, degp, W1, b1.reshape(1, H), W2, b2.reshape(1, H), Wc0)


def _tc_layer_body(p_ref, u_ref, degp_ref, b_ref, W_ref, o_ref):
    dis = _dis(degp_ref[...])
    agg = jnp.sum(p_ref[...], axis=0) + u_ref[...]
    h = jax.nn.relu(dis * agg + b_ref[...])
    o_ref[...] = dis * jnp.dot(h, W_ref[...], preferred_element_type=jnp.float32)


def _tc_layer(p, u, degp, b, W):
    return pl.pallas_call(
        _tc_layer_body,
        grid=(GRID,),
        in_specs=[
            pl.BlockSpec((2, BLK, H), lambda i: (0, i, 0)),
            pl.BlockSpec((BLK, H), lambda i: (i, 0)),
            pl.BlockSpec((2, BLK, H), lambda i: (0, i, 0)),
            pl.BlockSpec((1, H), lambda i: (0, 0)),
            pl.BlockSpec((H, H), lambda i: (0, 0)),
        ],
        out_specs=pl.BlockSpec((BLK, H), lambda i: (i, 0)),
        out_shape=jax.ShapeDtypeStruct((NPAD, H), jnp.float32),
    )(p, u, degp, b.reshape(1, H), W)


def _tc_final_body(p_ref, u_ref, degp_ref, b_ref, o_ref):
    dis = _dis(degp_ref[...])
    agg = jnp.sum(p_ref[...], axis=0) + u_ref[...]
    o_ref[...] = jax.nn.relu(dis * agg + b_ref[...])


def _tc_final(p, u, degp, b):
    return pl.pallas_call(
        _tc_final_body,
        grid=(GRID,),
        in_specs=[
            pl.BlockSpec((2, BLK, H), lambda i: (0, i, 0)),
            pl.BlockSpec((BLK, H), lambda i: (i, 0)),
            pl.BlockSpec((2, BLK, H), lambda i: (0, i, 0)),
            pl.BlockSpec((1, H), lambda i: (0, 0)),
        ],
        out_specs=pl.BlockSpec((BLK, H), lambda i: (i, 0)),
        out_shape=jax.ShapeDtypeStruct((NPAD, H), jnp.float32),
    )(p, u, degp, b.reshape(1, H))


def kernel(x, edge_index, W1, b1, W2, b2, Wc0, bc0, Wc1, bc1, Wc2, bc2):
    xp = jnp.zeros((NPAD, H), jnp.float32).at[:N].set(x)
    pad = EPAD - E
    rowp = jnp.concatenate([edge_index[0], jnp.zeros((pad,), jnp.int32)])
    colp = jnp.concatenate([edge_index[1], jnp.full((pad,), NPAD - 1, jnp.int32)])
    zerosNP = jnp.zeros((NPAD, H), jnp.float32)

    onesK = jnp.ones((K, H), jnp.float32)
    degp = _sc_deg(colp, onesK, zerosNP)

    u = _tc_enc(xp, degp, W1, b1, W2, b2, Wc0)
    p = _sc_scatter(u, rowp, colp, zerosNP)
    u = _tc_layer(p, u, degp, bc0, Wc1)
    p = _sc_scatter(u, rowp, colp, zerosNP)
    u = _tc_layer(p, u, degp, bc1, Wc2)
    p = _sc_scatter(u, rowp, colp, zerosNP)
    out = _tc_final(p, u, degp, bc2)
    return out[:N]


# speedup vs baseline: 1.0009x; 1.0009x over previous
import functools

import jax
import jax.numpy as jnp
from jax import lax
from jax.experimental import pallas as pl
from jax.experimental.pallas import tpu as pltpu
from jax.experimental.pallas import tpu_sc as plsc

N = 10000
H = 128
NPAD = 10240
E = 320000
K = 128
NTILES = 32
CHUNKS = 80
NH = 2
HC = CHUNKS // NH
EP_TILE = K * CHUNKS
EPAD = EP_TILE * NTILES
RPT = NPAD // 16
BLK = 512
GRID = NPAD // BLK

_MESH = dict(core_axis_name="c", subcore_axis_name="s")


def _sc_deg_body(col_hbm, ones_hbm, zeros_hbm, out_hbm, coli_v, ones_v, acc_sh):
    c = lax.axis_index("c")
    s = lax.axis_index("s")
    wid = s * 2 + c
    rbase = pl.multiple_of(s * RPT, RPT)

    pltpu.sync_copy(zeros_hbm.at[pl.ds(rbase, RPT)], acc_sh.at[pl.ds(rbase, RPT)])
    pltpu.sync_copy(ones_hbm, ones_v)
    plsc.subcore_barrier()

    ebase = pl.multiple_of(wid * EP_TILE, K)

    def body(j, carry):
        off = pl.multiple_of(ebase + j * K, K)
        pltpu.sync_copy(col_hbm.at[pl.ds(off, K)], coli_v)
        pltpu.sync_copy(ones_v, acc_sh.at[coli_v], add=True)
        return carry

    lax.fori_loop(0, CHUNKS, body, 0)
    plsc.subcore_barrier()
    pltpu.sync_copy(acc_sh.at[pl.ds(rbase, RPT)], out_hbm.at[c, pl.ds(rbase, RPT)])


_sc_deg = functools.partial(
    pl.kernel,
    mesh=plsc.VectorSubcoreMesh(**_MESH),
    out_type=jax.ShapeDtypeStruct((2, NPAD, H), jnp.float32),
    scratch_types=[
        pltpu.VMEM((K,), jnp.int32),
        pltpu.VMEM((K, H), jnp.float32),
        pltpu.VMEM_SHARED((NPAD, H), jnp.float32),
    ],
)(_sc_deg_body)


def _sc_scatter_body(u_hbm, row_hbm, col_hbm, zeros_hbm, out_hbm,
                     rowi0, rowi1, coli0, coli1, buf0, buf1, acc_sh,
                     gsem0):
    c = lax.axis_index("c")
    s = lax.axis_index("s")
    wid = s * 2 + c
    rbase = pl.multiple_of(s * RPT, RPT)

    pltpu.sync_copy(zeros_hbm.at[pl.ds(rbase, RPT)], acc_sh.at[pl.ds(rbase, RPT)])
    plsc.subcore_barrier()

    ebase = pl.multiple_of(wid * EP_TILE, K)

    def fire(jj, rowi, buf):
        pltpu.sync_copy(row_hbm.at[pl.ds(pl.multiple_of(ebase + jj * K, K), K)], rowi)
        pltpu.async_copy(u_hbm.at[rowi], buf, gsem0)

    def drain(rowi, buf):
        pltpu.make_async_copy(u_hbm.at[rowi], buf, gsem0).wait()

    fire(0, rowi0, buf0)

    def step(jj, rowi, coli, buf, nrowi, nbuf):
        drain(rowi, buf)

        @pl.when(jj + 1 < CHUNKS)
        def _():
            fire(jj + 1, nrowi, nbuf)

        pltpu.sync_copy(col_hbm.at[pl.ds(pl.multiple_of(ebase + jj * K, K), K)], coli)
        pltpu.sync_copy(buf, acc_sh.at[coli], add=True)

    def body(i, carry):
        j0 = pl.multiple_of(i * 2, 2)
        step(j0, rowi0, coli0, buf0, rowi1, buf1)
        step(j0 + 1, rowi1, coli1, buf1, rowi0, buf0)
        return carry

    lax.fori_loop(0, CHUNKS // 2, body, 0)
    plsc.subcore_barrier()
    pltpu.sync_copy(acc_sh.at[pl.ds(rbase, RPT)], out_hbm.at[c, pl.ds(rbase, RPT)])


_sc_scatter = functools.partial(
    pl.kernel,
    mesh=plsc.VectorSubcoreMesh(**_MESH),
    out_type=jax.ShapeDtypeStruct((2, NPAD, H), jnp.float32),
    scratch_types=[
        pltpu.VMEM((K,), jnp.int32),
        pltpu.VMEM((K,), jnp.int32),
        pltpu.VMEM((K,), jnp.int32),
        pltpu.VMEM((K,), jnp.int32),
        pltpu.VMEM((K, H), jnp.float32),
        pltpu.VMEM((K, H), jnp.float32),
        pltpu.VMEM_SHARED((NPAD, H), jnp.float32),
        pltpu.SemaphoreType.DMA,
    ],
)(_sc_scatter_body)


def _dis(degp_blk):
    d = jnp.sum(jnp.sum(degp_blk, axis=0), axis=1, keepdims=True) * (1.0 / H)
    return lax.rsqrt(1.0 + d)


def _tc_enc_body(x_ref, degp_ref, W1_ref, b1_ref, W2_ref, b2_ref, Wc_ref, u_ref):
    dis = _dis(degp_ref[...])
    h = jnp.dot(x_ref[...], W1_ref[...], preferred_element_type=jnp.float32)
    h = jax.nn.relu(h + b1_ref[...])
    h = jnp.dot(h, W2_ref[...], preferred_element_type=jnp.float32) + b2_ref[...]
    u_ref[...] = dis * jnp.dot(h, Wc_ref[...], preferred_element_type=jnp.float32)


def _tc_enc(xp, degp, W1, b1, W2, b2, Wc0):
    return pl.pallas_call(
        _tc_enc_body,
        grid=(GRID,),
        in_specs=[
            pl.BlockSpec((BLK, H), lambda i: (i, 0)),
            pl.BlockSpec((2, BLK, H), lambda i: (0, i, 0)),
            pl.BlockSpec((H, H), lambda i: (0, 0)),
            pl.BlockSpec((1, H), lambda i: (0, 0)),
            pl.BlockSpec((H, H), lambda i: (0, 0)),
            pl.BlockSpec((1, H), lambda i: (0, 0)),
            pl.BlockSpec((H, H), lambda i: (0, 0)),
        ],
        out_specs=pl.BlockSpec((BLK, H), lambda i: (i, 0)),
        out_shape=jax.ShapeDtypeStruct((NPAD, H), jnp.float32),
    )(xp, degp, W1, b1.reshape(1, H), W2, b2.reshape(1, H), Wc0)


def _tc_layer_body(p_ref, u_ref, degp_ref, b_ref, W_ref, o_ref):
    dis = _dis(degp_ref[...])
    agg = jnp.sum(p_ref[...], axis=0) + u_ref[...]
    h = jax.nn.relu(dis * agg + b_ref[...])
    o_ref[...] = dis * jnp.dot(h, W_ref[...], preferred_element_type=jnp.float32)


def _tc_layer(p, u, degp, b, W):
    return pl.pallas_call(
        _tc_layer_body,
        grid=(GRID,),
        in_specs=[
            pl.BlockSpec((2, BLK, H), lambda i: (0, i, 0)),
            pl.BlockSpec((BLK, H), lambda i: (i, 0)),
            pl.BlockSpec((2, BLK, H), lambda i: (0, i, 0)),
            pl.BlockSpec((1, H), lambda i: (0, 0)),
            pl.BlockSpec((H, H), lambda i: (0, 0)),
        ],
        out_specs=pl.BlockSpec((BLK, H), lambda i: (i, 0)),
        out_shape=jax.ShapeDtypeStruct((NPAD, H), jnp.float32),
    )(p, u, degp, b.reshape(1, H), W)


def _tc_final_body(p_ref, u_ref, degp_ref, b_ref, o_ref):
    dis = _dis(degp_ref[...])
    agg = jnp.sum(p_ref[...], axis=0) + u_ref[...]
    o_ref[...] = jax.nn.relu(dis * agg + b_ref[...])


def _tc_final(p, u, degp, b):
    return pl.pallas_call(
        _tc_final_body,
        grid=(GRID,),
        in_specs=[
            pl.BlockSpec((2, BLK, H), lambda i: (0, i, 0)),
            pl.BlockSpec((BLK, H), lambda i: (i, 0)),
            pl.BlockSpec((2, BLK, H), lambda i: (0, i, 0)),
            pl.BlockSpec((1, H), lambda i: (0, 0)),
        ],
        out_specs=pl.BlockSpec((BLK, H), lambda i: (i, 0)),
        out_shape=jax.ShapeDtypeStruct((NPAD, H), jnp.float32),
    )(p, u, degp, b.reshape(1, H))


def kernel(x, edge_index, W1, b1, W2, b2, Wc0, bc0, Wc1, bc1, Wc2, bc2):
    xp = jnp.zeros((NPAD, H), jnp.float32).at[:N].set(x)
    pad = EPAD - E
    padcol = N + (jnp.arange(pad, dtype=jnp.int32) % (NPAD - N))
    rowp = jnp.concatenate([edge_index[0], jnp.zeros((pad,), jnp.int32)])
    colp = jnp.concatenate([edge_index[1], padcol])
    zerosNP = jnp.zeros((NPAD, H), jnp.float32)

    onesK = jnp.ones((K, H), jnp.float32)
    degp = _sc_deg(colp, onesK, zerosNP)

    u = _tc_enc(xp, degp, W1, b1, W2, b2, Wc0)
    p = _sc_scatter(u, rowp, colp, zerosNP)
    u = _tc_layer(p, u, degp, bc0, Wc1)
    p = _sc_scatter(u, rowp, colp, zerosNP)
    u = _tc_layer(p, u, degp, bc1, Wc2)
    p = _sc_scatter(u, rowp, colp, zerosNP)
    out = _tc_final(p, u, degp, bc2)
    return out[:N]


# speedup vs baseline: 2.9592x; 2.9567x over previous
import functools

import jax
import jax.numpy as jnp
from jax import lax
from jax.experimental import pallas as pl
from jax.experimental.pallas import tpu as pltpu
from jax.experimental.pallas import tpu_sc as plsc

N = 10000
H = 128
NPAD = 10240
E = 320000
K = 128
NTILES = 32
CHUNKS = 80
NH = 2
HC = CHUNKS // NH
EP_TILE = K * CHUNKS
EPAD = EP_TILE * NTILES
RPT = NPAD // 16
CH0 = 60
BLK = 512
GRID = NPAD // BLK

_MESH = dict(core_axis_name="c", subcore_axis_name="s")


def _sc_deg_body(col_hbm, ones_hbm, zeros_hbm, out_hbm, coli_v, ones_v, acc_sh):
    c = lax.axis_index("c")
    s = lax.axis_index("s")
    wid = s * 2 + c
    rbase = pl.multiple_of(s * RPT, RPT)

    pltpu.sync_copy(zeros_hbm.at[pl.ds(rbase, RPT)], acc_sh.at[pl.ds(rbase, RPT)])
    pltpu.sync_copy(ones_hbm, ones_v)
    plsc.subcore_barrier()

    ebase = pl.multiple_of(wid * EP_TILE, K)

    def body(j, carry):
        off = pl.multiple_of(ebase + j * K, K)
        pltpu.sync_copy(col_hbm.at[pl.ds(off, K)], coli_v)
        pltpu.sync_copy(ones_v, acc_sh.at[coli_v], add=True)
        return carry

    lax.fori_loop(0, CHUNKS, body, 0)
    plsc.subcore_barrier()
    pltpu.sync_copy(acc_sh.at[pl.ds(rbase, RPT)], out_hbm.at[c, pl.ds(rbase, RPT)])


_sc_deg = functools.partial(
    pl.kernel,
    mesh=plsc.VectorSubcoreMesh(**_MESH),
    out_type=jax.ShapeDtypeStruct((2, NPAD, H), jnp.float32),
    scratch_types=[
        pltpu.VMEM((K,), jnp.int32),
        pltpu.VMEM((K, H), jnp.float32),
        pltpu.VMEM_SHARED((NPAD, H), jnp.float32),
    ],
)(_sc_deg_body)


def _sc_scatter_body(u_hbm, row_hbm, col_hbm, zeros_hbm, out_hbm,
                     rowi0, rowi1, coli0, coli1, buf0, buf1, acc_sh,
                     gsem0):
    c = lax.axis_index("c")
    s = lax.axis_index("s")
    wid = s * 2 + c
    rbase = pl.multiple_of(s * RPT, RPT)

    pltpu.sync_copy(zeros_hbm.at[pl.ds(rbase, RPT)], acc_sh.at[pl.ds(rbase, RPT)])
    plsc.subcore_barrier()

    ebase = pl.multiple_of((s * CHUNKS + c * CH0) * K, K)
    nch = CH0 - (2 * CH0 - CHUNKS) * c

    def fire(jj, rowi, buf):
        pltpu.sync_copy(row_hbm.at[pl.ds(pl.multiple_of(ebase + jj * K, K), K)], rowi)
        pltpu.async_copy(u_hbm.at[rowi], buf, gsem0)

    def drain(rowi, buf):
        pltpu.make_async_copy(u_hbm.at[rowi], buf, gsem0).wait()

    fire(0, rowi0, buf0)

    def step(jj, rowi, coli, buf, nrowi, nbuf):
        drain(rowi, buf)

        @pl.when(jj + 1 < nch)
        def _():
            fire(jj + 1, nrowi, nbuf)

        pltpu.sync_copy(col_hbm.at[pl.ds(pl.multiple_of(ebase + jj * K, K), K)], coli)
        pltpu.sync_copy(buf, acc_sh.at[coli], add=True)

    def body(i, carry):
        j0 = pl.multiple_of(i * 2, 2)
        step(j0, rowi0, coli0, buf0, rowi1, buf1)
        step(j0 + 1, rowi1, coli1, buf1, rowi0, buf0)
        return carry

    lax.fori_loop(0, nch // 2, body, 0)
    plsc.subcore_barrier()
    pltpu.sync_copy(acc_sh.at[pl.ds(rbase, RPT)], out_hbm.at[c, pl.ds(rbase, RPT)])


_sc_scatter = functools.partial(
    pl.kernel,
    mesh=plsc.VectorSubcoreMesh(**_MESH),
    out_type=jax.ShapeDtypeStruct((2, NPAD, H), jnp.float32),
    scratch_types=[
        pltpu.VMEM((K,), jnp.int32),
        pltpu.VMEM((K,), jnp.int32),
        pltpu.VMEM((K,), jnp.int32),
        pltpu.VMEM((K,), jnp.int32),
        pltpu.VMEM((K, H), jnp.float32),
        pltpu.VMEM((K, H), jnp.float32),
        pltpu.VMEM_SHARED((NPAD, H), jnp.float32),
        pltpu.SemaphoreType.DMA,
    ],
)(_sc_scatter_body)


def _dis(degp_blk):
    d = jnp.sum(jnp.sum(degp_blk, axis=0), axis=1, keepdims=True) * (1.0 / H)
    return lax.rsqrt(1.0 + d)


def _tc_enc_body(x_ref, degp_ref, W1_ref, b1_ref, W2_ref, b2_ref, Wc_ref, u_ref):
    dis = _dis(degp_ref[...])
    h = jnp.dot(x_ref[...], W1_ref[...], preferred_element_type=jnp.float32)
    h = jax.nn.relu(h + b1_ref[...])
    h = jnp.dot(h, W2_ref[...], preferred_element_type=jnp.float32) + b2_ref[...]
    u_ref[...] = dis * jnp.dot(h, Wc_ref[...], preferred_element_type=jnp.float32)


def _tc_enc(xp, degp, W1, b1, W2, b2, Wc0):
    return pl.pallas_call(
        _tc_enc_body,
        grid=(GRID,),
        in_specs=[
            pl.BlockSpec((BLK, H), lambda i: (i, 0)),
            pl.BlockSpec((2, BLK, H), lambda i: (0, i, 0)),
            pl.BlockSpec((H, H), lambda i: (0, 0)),
            pl.BlockSpec((1, H), lambda i: (0, 0)),
            pl.BlockSpec((H, H), lambda i: (0, 0)),
            pl.BlockSpec((1, H), lambda i: (0, 0)),
            pl.BlockSpec((H, H), lambda i: (0, 0)),
        ],
        out_specs=pl.BlockSpec((BLK, H), lambda i: (i, 0)),
        out_shape=jax.ShapeDtypeStruct((NPAD, H), jnp.float32),
    )(xp, degp, W1, b1.reshape(1, H), W2, b2.reshape(1, H), Wc0)


def _tc_layer_body(p_ref, u_ref, degp_ref, b_ref, W_ref, o_ref):
    dis = _dis(degp_ref[...])
    agg = jnp.sum(p_ref[...], axis=0) + u_ref[...]
    h = jax.nn.relu(dis * agg + b_ref[...])
    o_ref[...] = dis * jnp.dot(h, W_ref[...], preferred_element_type=jnp.float32)


def _tc_layer(p, u, degp, b, W):
    return pl.pallas_call(
        _tc_layer_body,
        grid=(GRID,),
        in_specs=[
            pl.BlockSpec((2, BLK, H), lambda i: (0, i, 0)),
            pl.BlockSpec((BLK, H), lambda i: (i, 0)),
            pl.BlockSpec((2, BLK, H), lambda i: (0, i, 0)),
            pl.BlockSpec((1, H), lambda i: (0, 0)),
            pl.BlockSpec((H, H), lambda i: (0, 0)),
        ],
        out_specs=pl.BlockSpec((BLK, H), lambda i: (i, 0)),
        out_shape=jax.ShapeDtypeStruct((NPAD, H), jnp.float32),
    )(p, u, degp, b.reshape(1, H), W)


def _tc_final_body(p_ref, u_ref, degp_ref, b_ref, o_ref):
    dis = _dis(degp_ref[...])
    agg = jnp.sum(p_ref[...], axis=0) + u_ref[...]
    o_ref[...] = jax.nn.relu(dis * agg + b_ref[...])


def _tc_final(p, u, degp, b):
    return pl.pallas_call(
        _tc_final_body,
        grid=(GRID,),
        in_specs=[
            pl.BlockSpec((2, BLK, H), lambda i: (0, i, 0)),
            pl.BlockSpec((BLK, H), lambda i: (i, 0)),
            pl.BlockSpec((2, BLK, H), lambda i: (0, i, 0)),
            pl.BlockSpec((1, H), lambda i: (0, 0)),
        ],
        out_specs=pl.BlockSpec((BLK, H), lambda i: (i, 0)),
        out_shape=jax.ShapeDtypeStruct((NPAD, H), jnp.float32),
    )(p, u, degp, b.reshape(1, H))


def kernel(x, edge_index, W1, b1, W2, b2, Wc0, bc0, Wc1, bc1, Wc2, bc2):
    xp = jnp.zeros((NPAD, H), jnp.float32).at[:N].set(x)
    pad = EPAD - E
    padcol = N + (jnp.arange(pad, dtype=jnp.int32) % (NPAD - N))
    rowp = jnp.concatenate([edge_index[0], jnp.zeros((pad,), jnp.int32)])
    colp = jnp.concatenate([edge_index[1], padcol])
    zerosNP = jnp.zeros((NPAD, H), jnp.float32)

    onesK = jnp.ones((K, H), jnp.float32)
    degp = _sc_deg(colp, onesK, zerosNP)

    u = _tc_enc(xp, degp, W1, b1, W2, b2, Wc0)
    p = _sc_scatter(u, rowp, colp, zerosNP)
    u = _tc_layer(p, u, degp, bc0, Wc1)
    p = _sc_scatter(u, rowp, colp, zerosNP)
    u = _tc_layer(p, u, degp, bc1, Wc2)
    p = _sc_scatter(u, rowp, colp, zerosNP)
    out = _tc_final(p, u, degp, bc2)
    return out[:N]
